# baseline (speedup 1.0000x reference)
import functools

import jax
import jax.numpy as jnp
from jax import lax
from jax.experimental import pallas as pl
from jax.experimental.pallas import tpu as pltpu
from jax.experimental.pallas import tpu_sc as plsc

NNZ = 1638400
INPUT_SIZE = 1000000
EPS = 1e-05

NW = 32
SHARD = 32768
FPAD = NW * SHARD
PER_W = NNZ // NW
PER_LANE = PER_W // 16

CAP = 256
MAIN_WORDS = NW * NW * 16 * CAP
OVF_CAP = PER_LANE
OVF_WORDS = NW * 16 * OVF_CAP
BIG = MAIN_WORDS + OVF_WORDS

K1_CH = 12800
K3_CH = 5120

_mesh = plsc.VectorSubcoreMesh(core_axis_name="c", subcore_axis_name="s")

_GATHER_DNUMS = lax.GatherDimensionNumbers(
    offset_dims=(), collapsed_slice_dims=(0,), start_index_map=(0,))


def _lane_iota():
    return lax.iota(jnp.int32, 16)


def _perm(x, inds):
    return lax.gather(x, inds[:, None], _GATHER_DNUMS, (1,),
                      mode=lax.GatherScatterMode.PROMISE_IN_BOUNDS)


def _shift_up(x, k, iota):
    return _perm(x, jnp.maximum(iota - k, 0))


def _scatter_max_vreg(acc_ref, li, vv, valid, iota):
    kc = jnp.bitwise_and(li, SHARD - 1)
    old = plsc.load_gather(acc_ref, [kc])
    nv = jnp.maximum(jnp.where(valid, vv, jnp.float32(-1.0)), old)
    plsc.store_scatter(acc_ref, [kc], nv, mask=valid)
    got = plsc.load_gather(acc_ref, [kc])
    lost = valid & (got < nv)

    @pl.when(jnp.any(lost))
    def _():
        _seg_scatter_max(acc_ref, li, vv, valid, iota)


def _seg_scatter_max(acc_ref, li, vv, valid, iota):
    key = jnp.where(valid, li, jnp.int32(SHARD))
    val = jnp.where(valid, vv, jnp.float32(-1.0))
    ks, vs = plsc.sort_key_val(key, val)
    for sh in (1, 2, 4, 8):
        pk = _shift_up(ks, sh, iota)
        pv = _shift_up(vs, sh, iota)
        take = (pk == ks) & (iota >= sh)
        vs = jnp.maximum(vs, jnp.where(take, pv, jnp.float32(-1.0)))
    nk = _perm(ks, jnp.minimum(iota + 1, 15))
    is_last = (nk != ks) | (iota == 15)
    do_store = is_last & (ks < SHARD)
    kc = jnp.bitwise_and(ks, SHARD - 1)
    old = plsc.load_gather(acc_ref, [kc])
    plsc.store_scatter(acc_ref, [kc], jnp.maximum(vs, old), mask=do_store)


def _k1_body(idx_hbm, val_hbm, bidx_hbm, bval_hbm, cnt_hbm, ocnt_hbm,
             idx_st, val_st, slot_st, cur_ref, ocur_ref, sem1, sem2):
    w = lax.axis_index("s") * 2 + lax.axis_index("c")
    iota = _lane_iota()
    base_lane = iota * CAP + w * (16 * CAP)
    base_ovf = MAIN_WORDS + w * (16 * OVF_CAP) + iota * OVF_CAP

    def zero_row(d, _):
        cur_ref[pl.ds(d * 16, 16)] = jnp.zeros((16,), jnp.int32)
        return 0
    lax.fori_loop(0, NW, zero_row, 0)
    ocur_ref[:] = jnp.zeros((16,), jnp.int32)

    def chunk(ci, _):
        ebase = w * PER_W + ci * K1_CH
        pltpu.sync_copy(idx_hbm.at[pl.ds(ebase, K1_CH)], idx_st)
        pltpu.sync_copy(val_hbm.at[pl.ds(ebase, K1_CH)], val_st)

        def vreg(j, _):
            iv = idx_st[pl.ds(j * 16, 16)]
            vv = val_st[pl.ds(j * 16, 16)]
            dest = lax.shift_right_logical(iv, 15)
            fl = dest * 16 + iota
            pos = plsc.load_gather(cur_ref, [fl])
            plsc.store_scatter(cur_ref, [fl], pos + 1)
            ovf = pos >= CAP
            opos = ocur_ref[:]
            ocur_ref[:] = opos + ovf.astype(jnp.int32)
            slot_main = dest * (NW * 16 * CAP) + base_lane + pos
            slot = jnp.where(ovf, base_ovf + opos, slot_main)
            row = lax.shift_right_logical(j, 3)
            col = jnp.bitwise_and(j, 7) * 16
            slot_st[row, pl.ds(col, 16)] = slot
            val_st[pl.ds(j * 16, 16)] = jnp.abs(vv)
            return 0
        lax.fori_loop(0, K1_CH // 16, vreg, 0, unroll=2)

        def fire(r, _):
            pltpu.make_async_copy(idx_st.at[pl.ds(r * 128, 128)],
                                  bidx_hbm.at[slot_st.at[r]], sem1).start()
            pltpu.make_async_copy(val_st.at[pl.ds(r * 128, 128)],
                                  bval_hbm.at[slot_st.at[r]], sem2).start()
            return 0
        lax.fori_loop(0, K1_CH // 128, fire, 0)
        pltpu.make_async_copy(idx_hbm.at[pl.ds(0, K1_CH)], idx_st, sem1).wait()
        pltpu.make_async_copy(val_hbm.at[pl.ds(0, K1_CH)], val_st, sem2).wait()
        return 0
    lax.fori_loop(0, PER_W // K1_CH, chunk, 0)

    pltpu.sync_copy(cur_ref, cnt_hbm.at[pl.ds(w * NW * 16, NW * 16)])
    pltpu.sync_copy(ocur_ref, ocnt_hbm.at[pl.ds(w * 16, 16)])


@functools.partial(jax.jit, donate_argnums=())
def _k1(idx, val):
    return pl.kernel(
        _k1_body,
        out_type=(
            jax.ShapeDtypeStruct((BIG,), jnp.int32),
            jax.ShapeDtypeStruct((BIG,), jnp.float32),
            jax.ShapeDtypeStruct((NW * NW * 16,), jnp.int32),
            jax.ShapeDtypeStruct((NW * 16,), jnp.int32),
        ),
        mesh=_mesh,
        compiler_params=pltpu.CompilerParams(needs_layout_passes=False),
        scratch_types=[
            pltpu.VMEM((K1_CH,), jnp.int32),
            pltpu.VMEM((K1_CH,), jnp.float32),
            pltpu.VMEM((K1_CH // 128, 128), jnp.int32),
            pltpu.VMEM((NW * 16,), jnp.int32),
            pltpu.VMEM((16,), jnp.int32),
            pltpu.SemaphoreType.DMA,
            pltpu.SemaphoreType.DMA,
        ],
    )(idx, val)


def _k2_body(bidx_hbm, bval_hbm, cnt_hbm, ocnt_hbm, maxpad_hbm, newmax_hbm,
             acc_ref, cst_ref, ocst_ref, bidx_st, bval_st, oidx_st, oval_st):
    w = lax.axis_index("s") * 2 + lax.axis_index("c")
    iota = _lane_iota()

    pltpu.sync_copy(maxpad_hbm.at[pl.ds(w * SHARD, SHARD)], acc_ref)
    pltpu.sync_copy(cnt_hbm, cst_ref)
    pltpu.sync_copy(ocnt_hbm, ocst_ref)

    def src_loop(s, _):
        blk = (w * NW + s) * (16 * CAP)
        pltpu.sync_copy(bidx_hbm.at[pl.ds(blk, 16 * CAP)], bidx_st)
        pltpu.sync_copy(bval_hbm.at[pl.ds(blk, 16 * CAP)], bval_st)
        cvec = cst_ref[pl.ds((s * NW + w) * 16, 16)]

        def lane_loop(l, _):
            c = jnp.sum(jnp.where(iota == l, cvec, 0))
            c = jnp.minimum(c, CAP)
            trips = lax.shift_right_logical(c + 15, 4)

            def k_loop(k, _):
                off = l * CAP + k * 16
                ivec = bidx_st[pl.ds(off, 16)]
                vvec = bval_st[pl.ds(off, 16)]
                valid = (k * 16 + iota) < c
                li = jnp.bitwise_and(ivec, SHARD - 1)
                _scatter_max_vreg(acc_ref, li, vvec, valid, iota)
                return 0
            lax.fori_loop(0, trips, k_loop, 0)
            return 0
        lax.fori_loop(0, 16, lane_loop, 0)
        return 0
    lax.fori_loop(0, NW, src_loop, 0)

    def ovf_src(s, _):
        ovec_all = ocst_ref[pl.ds(s * 16, 16)]

        def ovf_lane(l, _):
            oc = jnp.sum(jnp.where(iota == l, ovec_all, 0))
            oc = jnp.minimum(oc, OVF_CAP)

            @pl.when(oc > 0)
            def _():
                obase = (s * 16 + l) * OVF_CAP
                pltpu.sync_copy(bidx_hbm.at[pl.ds(MAIN_WORDS + obase, OVF_CAP)],
                                oidx_st)
                pltpu.sync_copy(bval_hbm.at[pl.ds(MAIN_WORDS + obase, OVF_CAP)],
                                oval_st)
                trips = lax.shift_right_logical(oc + 15, 4)

                def k_loop(k, _):
                    ivec = oidx_st[pl.ds(k * 16, 16)]
                    vvec = oval_st[pl.ds(k * 16, 16)]
                    dest = lax.shift_right_logical(ivec, 15)
                    valid = ((k * 16 + iota) < oc) & (dest == w)
                    li = jnp.bitwise_and(ivec, SHARD - 1)
                    _scatter_max_vreg(acc_ref, li, vvec, valid, iota)
                    return 0
                lax.fori_loop(0, trips, k_loop, 0)
            return 0
        lax.fori_loop(0, 16, ovf_lane, 0)
        return 0
    lax.fori_loop(0, NW, ovf_src, 0)

    pltpu.sync_copy(acc_ref, newmax_hbm.at[pl.ds(w * SHARD, SHARD)])


@jax.jit
def _k2(bidx, bval, cnt, ocnt, maxpad):
    return pl.kernel(
        _k2_body,
        out_type=jax.ShapeDtypeStruct((FPAD,), jnp.float32),
        mesh=_mesh,
        compiler_params=pltpu.CompilerParams(needs_layout_passes=False),
        scratch_types=[
            pltpu.VMEM((SHARD,), jnp.float32),
            pltpu.VMEM((NW * NW * 16,), jnp.int32),
            pltpu.VMEM((NW * 16,), jnp.int32),
            pltpu.VMEM((16 * CAP,), jnp.int32),
            pltpu.VMEM((16 * CAP,), jnp.float32),
            pltpu.VMEM((OVF_CAP,), jnp.int32),
            pltpu.VMEM((OVF_CAP,), jnp.float32),
        ],
    )(bidx, bval, cnt, ocnt, maxpad)


def _k3_body(idx2d_hbm, val_hbm, newmax_hbm, bias_hbm, out_hbm,
             idxr_st, val_st, g_st, b_st, out_st, sem1, sem2):
    w = lax.axis_index("s") * 2 + lax.axis_index("c")
    iota = _lane_iota()
    nrows = K3_CH // 128

    def chunk(ci, _):
        ebase = w * PER_W + ci * K3_CH
        rbase = pl.multiple_of(ebase // 128, 8)
        pltpu.sync_copy(idx2d_hbm.at[pl.ds(rbase, nrows)], idxr_st)
        pltpu.sync_copy(val_hbm.at[pl.ds(ebase, K3_CH)], val_st)

        def fire(r, _):
            pltpu.make_async_copy(newmax_hbm.at[idxr_st.at[r]],
                                  g_st.at[pl.ds(r * 128, 128)], sem1).start()
            pltpu.make_async_copy(bias_hbm.at[idxr_st.at[r]],
                                  b_st.at[pl.ds(r * 128, 128)], sem2).start()
            return 0
        lax.fori_loop(0, nrows, fire, 0)
        pltpu.make_async_copy(newmax_hbm.at[pl.ds(0, K3_CH)], g_st, sem1).wait()
        pltpu.make_async_copy(bias_hbm.at[pl.ds(0, K3_CH)], b_st, sem2).wait()

        def vreg(j, _):
            sl = pl.ds(j * 16, 16)
            v = val_st[sl]
            g = g_st[sl]
            b = b_st[sl]
            denom = jnp.maximum(g, jnp.float32(EPS))
            q = v / denom
            q = jnp.minimum(jnp.maximum(q, jnp.float32(-1.0)), jnp.float32(1.0))
            out_st[sl] = q + b
            return 0
        lax.fori_loop(0, K3_CH // 16, vreg, 0, unroll=2)

        pltpu.sync_copy(out_st, out_hbm.at[pl.ds(ebase, K3_CH)])
        return 0
    lax.fori_loop(0, PER_W // K3_CH, chunk, 0)
    del iota


@jax.jit
def _k3(idx2d, val, newmax, bias):
    return pl.kernel(
        _k3_body,
        out_type=jax.ShapeDtypeStruct((NNZ,), jnp.float32),
        mesh=_mesh,
        compiler_params=pltpu.CompilerParams(needs_layout_passes=False),
        scratch_types=[
            pltpu.VMEM((K3_CH // 128, 128), jnp.int32),
            pltpu.VMEM((K3_CH,), jnp.float32),
            pltpu.VMEM((K3_CH,), jnp.float32),
            pltpu.VMEM((K3_CH,), jnp.float32),
            pltpu.VMEM((K3_CH,), jnp.float32),
            pltpu.SemaphoreType.DMA,
            pltpu.SemaphoreType.DMA,
        ],
    )(idx2d, val, newmax, bias)


def kernel(values_x, max_x, bias_x, indices_x):
    idx = indices_x.astype(jnp.int32)
    maxpad = jnp.concatenate(
        [max_x, jnp.zeros((FPAD - INPUT_SIZE,), jnp.float32)])
    bidx, bval, cnt, ocnt = _k1(idx, values_x)
    newmax = _k2(bidx, bval, cnt, ocnt, maxpad)
    return _k3(idx.reshape(NNZ // 128, 128), values_x, newmax, bias_x)


# speedup vs baseline: 6.4102x; 6.4102x over previous
import functools

import jax
import jax.numpy as jnp
from jax import lax
from jax.experimental import pallas as pl
from jax.experimental.pallas import tpu as pltpu
from jax.experimental.pallas import tpu_sc as plsc

NNZ = 1638400
INPUT_SIZE = 1000000
EPS = 1e-05

NW = 32
SHARD = 32768
FPAD = NW * SHARD
PER_W = NNZ // NW
PER_LANE = PER_W // 16

CAP = 152
HALF = NW * 16 * 16 * CAP
SC_WORDS = HALF + 16
OVF_CAP = PER_LANE
OVF_WORDS = NW * 16 * OVF_CAP

K1_CH = 12800
K3_CH = 5120

_mesh = plsc.VectorSubcoreMesh(core_axis_name="c", subcore_axis_name="s")

_GATHER_DNUMS = lax.GatherDimensionNumbers(
    offset_dims=(), collapsed_slice_dims=(0,), start_index_map=(0,))


def _lane_iota():
    return lax.iota(jnp.int32, 16)


def _perm(x, inds):
    return lax.gather(x, inds[:, None], _GATHER_DNUMS, (1,),
                      mode=lax.GatherScatterMode.PROMISE_IN_BOUNDS)


def _shift_up(x, k, iota):
    return _perm(x, jnp.maximum(iota - k, 0))


def _scatter_max_vreg(acc_ref, li, vv, valid, iota):
    kc = jnp.bitwise_and(li, SHARD - 1)
    old = plsc.load_gather(acc_ref, [kc])
    nv = jnp.maximum(jnp.where(valid, vv, jnp.float32(-1.0)), old)
    plsc.store_scatter(acc_ref, [kc], nv, mask=valid)
    got = plsc.load_gather(acc_ref, [kc])
    lost = valid & (got < nv)

    @pl.when(jnp.any(lost))
    def _():
        _seg_scatter_max(acc_ref, li, vv, valid, iota)


def _seg_scatter_max(acc_ref, li, vv, valid, iota):
    key = jnp.where(valid, li, jnp.int32(SHARD))
    val = jnp.where(valid, vv, jnp.float32(-1.0))
    ks, vs = plsc.sort_key_val(key, val)
    for sh in (1, 2, 4, 8):
        pk = _shift_up(ks, sh, iota)
        pv = _shift_up(vs, sh, iota)
        take = (pk == ks) & (iota >= sh)
        vs = jnp.maximum(vs, jnp.where(take, pv, jnp.float32(-1.0)))
    nk = _perm(ks, jnp.minimum(iota + 1, 15))
    is_last = (nk != ks) | (iota == 15)
    do_store = is_last & (ks < SHARD)
    kc = jnp.bitwise_and(ks, SHARD - 1)
    old = plsc.load_gather(acc_ref, [kc])
    plsc.store_scatter(acc_ref, [kc], jnp.maximum(vs, old), mask=do_store)


def _k1_body(idx_hbm, val_hbm, bbuck_hbm, cnt_hbm, ocnt_hbm, oidx_hbm,
             oval_hbm, idx_st, val_st, pk_st, slot_st, cur_ref, ocur_ref,
             ovslot_ref, spk_ref, sem1):
    ss = lax.axis_index("s")
    cc = lax.axis_index("c")
    w = ss * 2 + cc
    iota = _lane_iota()
    base_ovf = w * (16 * OVF_CAP) + iota * OVF_CAP

    def zero_row(d, _):
        cur_ref[pl.ds(d * 16, 16)] = jnp.zeros((16,), jnp.int32)
        return 0
    lax.fori_loop(0, NW, zero_row, 0)
    ocur_ref[:] = jnp.zeros((16,), jnp.int32)

    def chunk(ci, _):
        ebase = w * PER_W + ci * K1_CH
        pltpu.sync_copy(idx_hbm.at[pl.ds(ebase, K1_CH)], idx_st)
        pltpu.sync_copy(val_hbm.at[pl.ds(ebase, K1_CH)], val_st)

        def vreg(j, _):
            iv = idx_st[pl.ds(j * 16, 16)]
            vv = val_st[pl.ds(j * 16, 16)]
            dest = lax.shift_right_logical(iv, 15)
            fl = dest * 16 + iota
            pos = plsc.load_gather(cur_ref, [fl])
            plsc.store_scatter(cur_ref, [fl], pos + 1)
            av = jnp.abs(vv)
            bits = lax.bitcast_convert_type(av, jnp.int32)
            rnd = lax.shift_right_logical(
                bits + 0x7FFF + jnp.bitwise_and(
                    lax.shift_right_logical(bits, 16), 1), 16)
            li = jnp.bitwise_and(iv, SHARD - 1)
            word = jnp.bitwise_or(lax.shift_left(li, 16), rnd)
            pk_st[pl.ds(j * 16, 16)] = word
            ovf = pos >= CAP
            slot_main = ((dest * 16 + ss) * 16) * CAP + iota * CAP + pos
            slot = jnp.where(ovf, HALF + iota, slot_main)
            row = lax.shift_right_logical(j, 3)
            col = jnp.bitwise_and(j, 7) * 16
            slot_st[row, pl.ds(col, 16)] = slot

            @pl.when(jnp.any(ovf))
            def _():
                opos = ocur_ref[:]
                ocur_ref[:] = opos + ovf.astype(jnp.int32)
                oslot = jnp.where(ovf, base_ovf + opos, OVF_WORDS + iota)
                ovslot_ref[:] = oslot
                val_st[pl.ds(j * 16, 16)] = av
                pltpu.sync_copy(idx_st.at[pl.ds(j * 16, 16)],
                                oidx_hbm.at[ovslot_ref])
                pltpu.sync_copy(val_st.at[pl.ds(j * 16, 16)],
                                oval_hbm.at[ovslot_ref])
            return 0
        lax.fori_loop(0, K1_CH // 16, vreg, 0, unroll=2)

        def fire(r, _):
            pltpu.make_async_copy(pk_st.at[pl.ds(r * 128, 128)],
                                  spk_ref.at[slot_st.at[r]], sem1).start()
            return 0
        lax.fori_loop(0, K1_CH // 128, fire, 0)
        pltpu.make_async_copy(idx_hbm.at[pl.ds(0, K1_CH)], pk_st, sem1).wait()
        return 0
    lax.fori_loop(0, PER_W // K1_CH, chunk, 0)

    plsc.subcore_barrier()
    def flush(d, _):
        off = ((d * 16 + ss) * 16) * CAP
        pltpu.make_async_copy(spk_ref.at[pl.ds(off, 16 * CAP)],
                              bbuck_hbm.at[pl.ds(cc * HALF + off, 16 * CAP)],
                              sem1).start()
        return 0
    lax.fori_loop(0, NW, flush, 0)
    pltpu.make_async_copy(idx_hbm.at[pl.ds(0, NW * 16 * CAP)],
                          spk_ref.at[pl.ds(0, NW * 16 * CAP)], sem1).wait()

    pltpu.sync_copy(cur_ref, cnt_hbm.at[pl.ds(w * NW * 16, NW * 16)])
    pltpu.sync_copy(ocur_ref, ocnt_hbm.at[pl.ds(w * 16, 16)])


@functools.partial(jax.jit, donate_argnums=())
def _k1(idx, val):
    return pl.kernel(
        _k1_body,
        out_type=(
            jax.ShapeDtypeStruct((2 * HALF,), jnp.int32),
            jax.ShapeDtypeStruct((NW * NW * 16,), jnp.int32),
            jax.ShapeDtypeStruct((NW * 16,), jnp.int32),
            jax.ShapeDtypeStruct((OVF_WORDS + 16,), jnp.int32),
            jax.ShapeDtypeStruct((OVF_WORDS + 16,), jnp.float32),
        ),
        mesh=_mesh,
        compiler_params=pltpu.CompilerParams(needs_layout_passes=False),
        scratch_types=[
            pltpu.VMEM((K1_CH,), jnp.int32),
            pltpu.VMEM((K1_CH,), jnp.float32),
            pltpu.VMEM((K1_CH,), jnp.int32),
            pltpu.VMEM((K1_CH // 128, 128), jnp.int32),
            pltpu.VMEM((NW * 16,), jnp.int32),
            pltpu.VMEM((16,), jnp.int32),
            pltpu.VMEM((16,), jnp.int32),
            pltpu.VMEM_SHARED((SC_WORDS,), jnp.int32),
            pltpu.SemaphoreType.DMA,
        ],
    )(idx, val)


def _k2_body(bbuck_hbm, cnt_hbm, ocnt_hbm, oidx_hbm, oval_hbm, maxpad_hbm,
             newmax_hbm, acc_ref, cst_ref, ocst_ref, pk_st, oidx_st, oval_st):
    w = lax.axis_index("s") * 2 + lax.axis_index("c")
    iota = _lane_iota()

    pltpu.sync_copy(maxpad_hbm.at[pl.ds(w * SHARD, SHARD)], acc_ref)
    pltpu.sync_copy(cnt_hbm, cst_ref)
    pltpu.sync_copy(ocnt_hbm, ocst_ref)

    def src_loop(sg, _):
        sc = jnp.bitwise_and(sg, 1)
        s16 = lax.shift_right_logical(sg, 1)
        blk = sc * HALF + ((w * 16 + s16) * 16) * CAP
        pltpu.sync_copy(bbuck_hbm.at[pl.ds(blk, 16 * CAP)], pk_st)
        cvec = cst_ref[pl.ds((sg * NW + w) * 16, 16)]

        def lane_loop(l, _):
            c = jnp.sum(jnp.where(iota == l, cvec, 0))
            c = jnp.minimum(c, CAP)
            trips = lax.shift_right_logical(c + 15, 4)

            def k_loop(k, _):
                off = l * CAP + k * 16
                word = pk_st[pl.ds(off, 16)]
                li = lax.shift_right_logical(word, 16)
                vvec = lax.bitcast_convert_type(
                    lax.shift_left(word, 16), jnp.float32)
                valid = (k * 16 + iota) < c
                _scatter_max_vreg(acc_ref, li, vvec, valid, iota)
                return 0
            lax.fori_loop(0, trips, k_loop, 0)
            return 0
        lax.fori_loop(0, 16, lane_loop, 0)
        return 0
    lax.fori_loop(0, NW, src_loop, 0)

    def ovf_src(s, _):
        ovec_all = ocst_ref[pl.ds(s * 16, 16)]

        def ovf_lane(l, _):
            oc = jnp.sum(jnp.where(iota == l, ovec_all, 0))
            oc = jnp.minimum(oc, OVF_CAP)

            @pl.when(oc > 0)
            def _():
                obase = (s * 16 + l) * OVF_CAP
                pltpu.sync_copy(oidx_hbm.at[pl.ds(obase, OVF_CAP)], oidx_st)
                pltpu.sync_copy(oval_hbm.at[pl.ds(obase, OVF_CAP)], oval_st)
                trips = lax.shift_right_logical(oc + 15, 4)

                def k_loop(k, _):
                    ivec = oidx_st[pl.ds(k * 16, 16)]
                    vvec = oval_st[pl.ds(k * 16, 16)]
                    dest = lax.shift_right_logical(ivec, 15)
                    valid = ((k * 16 + iota) < oc) & (dest == w)
                    li = jnp.bitwise_and(ivec, SHARD - 1)
                    _scatter_max_vreg(acc_ref, li, vvec, valid, iota)
                    return 0
                lax.fori_loop(0, trips, k_loop, 0)
            return 0
        lax.fori_loop(0, 16, ovf_lane, 0)
        return 0
    lax.fori_loop(0, NW, ovf_src, 0)

    pltpu.sync_copy(acc_ref, newmax_hbm.at[pl.ds(w * SHARD, SHARD)])


@jax.jit
def _k2(bbuck, cnt, ocnt, oidx, oval, maxpad):
    return pl.kernel(
        _k2_body,
        out_type=jax.ShapeDtypeStruct((FPAD,), jnp.float32),
        mesh=_mesh,
        compiler_params=pltpu.CompilerParams(needs_layout_passes=False),
        scratch_types=[
            pltpu.VMEM((SHARD,), jnp.float32),
            pltpu.VMEM((NW * NW * 16,), jnp.int32),
            pltpu.VMEM((NW * 16,), jnp.int32),
            pltpu.VMEM((16 * CAP,), jnp.int32),
            pltpu.VMEM((OVF_CAP,), jnp.int32),
            pltpu.VMEM((OVF_CAP,), jnp.float32),
        ],
    )(bbuck, cnt, ocnt, oidx, oval, maxpad)


def _k3_body(idx2d_hbm, val_hbm, newmax_hbm, bias_hbm, out_hbm,
             idxr_st, val_st, g_st, b_st, out_st, sem1, sem2):
    w = lax.axis_index("s") * 2 + lax.axis_index("c")
    iota = _lane_iota()
    nrows = K3_CH // 128

    def chunk(ci, _):
        ebase = w * PER_W + ci * K3_CH
        rbase = pl.multiple_of(ebase // 128, 8)
        pltpu.sync_copy(idx2d_hbm.at[pl.ds(rbase, nrows)], idxr_st)
        pltpu.sync_copy(val_hbm.at[pl.ds(ebase, K3_CH)], val_st)

        def fire(r, _):
            pltpu.make_async_copy(newmax_hbm.at[idxr_st.at[r]],
                                  g_st.at[pl.ds(r * 128, 128)], sem1).start()
            pltpu.make_async_copy(bias_hbm.at[idxr_st.at[r]],
                                  b_st.at[pl.ds(r * 128, 128)], sem2).start()
            return 0
        lax.fori_loop(0, nrows, fire, 0)
        pltpu.make_async_copy(newmax_hbm.at[pl.ds(0, K3_CH)], g_st, sem1).wait()
        pltpu.make_async_copy(bias_hbm.at[pl.ds(0, K3_CH)], b_st, sem2).wait()

        def vreg(j, _):
            sl = pl.ds(j * 16, 16)
            v = val_st[sl]
            g = g_st[sl]
            b = b_st[sl]
            denom = jnp.maximum(g, jnp.float32(EPS))
            q = v / denom
            q = jnp.minimum(jnp.maximum(q, jnp.float32(-1.0)), jnp.float32(1.0))
            out_st[sl] = q + b
            return 0
        lax.fori_loop(0, K3_CH // 16, vreg, 0, unroll=2)

        pltpu.sync_copy(out_st, out_hbm.at[pl.ds(ebase, K3_CH)])
        return 0
    lax.fori_loop(0, PER_W // K3_CH, chunk, 0)
    del iota


@jax.jit
def _k3(idx2d, val, newmax, bias):
    return pl.kernel(
        _k3_body,
        out_type=jax.ShapeDtypeStruct((NNZ,), jnp.float32),
        mesh=_mesh,
        compiler_params=pltpu.CompilerParams(needs_layout_passes=False),
        scratch_types=[
            pltpu.VMEM((K3_CH // 128, 128), jnp.int32),
            pltpu.VMEM((K3_CH,), jnp.float32),
            pltpu.VMEM((K3_CH,), jnp.float32),
            pltpu.VMEM((K3_CH,), jnp.float32),
            pltpu.VMEM((K3_CH,), jnp.float32),
            pltpu.SemaphoreType.DMA,
            pltpu.SemaphoreType.DMA,
        ],
    )(idx2d, val, newmax, bias)


def kernel(values_x, max_x, bias_x, indices_x):
    idx = indices_x.astype(jnp.int32)
    maxpad = jnp.concatenate(
        [max_x, jnp.zeros((FPAD - INPUT_SIZE,), jnp.float32)])
    bbuck, cnt, ocnt, oidx, oval = _k1(idx, values_x)
    newmax = _k2(bbuck, cnt, ocnt, oidx, oval, maxpad)
    return _k3(idx.reshape(NNZ // 128, 128), values_x, newmax, bias_x)
